# Initial kernel scaffold; baseline (speedup 1.0000x reference)
import functools

import jax
import jax.numpy as jnp
from jax import lax
from jax.experimental import pallas as pl
from jax.experimental.pallas import tpu as pltpu
from jax.experimental.pallas import tpu_sc as plsc

_N = 10000
_D = 128
_E = 320000
_C = 128
_ROWS = _E // _C
_NC = 2
_NS = 16
_NW = _NC * _NS
_RPW = _ROWS // _NW
_EXTRA = _ROWS - _RPW * _NW
_ZCH = 80
_LANES = 8


def _layer_body(h_hbm, src_hbm, dst_hbm, w_hbm, out_hbm,
                partial, gbuf, zbuf, src_v, dst_v, w_v):
    c = lax.axis_index("c")
    s = lax.axis_index("s")
    wid = s * _NC + c

    def _zrow(i, carry):
        for j in range(_LANES):
            zbuf[i, pl.ds(j * 16, 16)] = jnp.zeros((16,), jnp.float32)
        return carry
    lax.fori_loop(0, _ZCH, _zrow, 0)

    zbase = s * 640
    nz = jnp.where(s < _NS - 1, 8, 5)

    def _zchunk(k, carry):
        pltpu.sync_copy(zbuf, partial.at[pl.ds(zbase + k * _ZCH, _ZCH), :])
        return carry
    lax.fori_loop(0, nz, _zchunk, 0)

    plsc.subcore_barrier()

    nrows = _RPW + jnp.where(wid < _EXTRA, 1, 0)
    rbase = wid * _RPW + jnp.minimum(wid, _EXTRA)

    def _erow(r, carry):
        row = rbase + r
        pltpu.sync_copy(src_hbm.at[row], src_v)
        pltpu.sync_copy(dst_hbm.at[row], dst_v)
        pltpu.sync_copy(w_hbm.at[row], w_v)
        pltpu.sync_copy(h_hbm.at[src_v], gbuf)

        def _scale(e, carry2):
            w = w_v[e]
            for j in range(_LANES):
                gbuf[e, pl.ds(j * 16, 16)] = gbuf[e, pl.ds(j * 16, 16)] * w
            return carry2
        lax.fori_loop(0, _C, _scale, 0)

        pltpu.sync_copy(gbuf, partial.at[dst_v], add=True)
        return carry
    lax.fori_loop(0, nrows, _erow, 0)

    plsc.subcore_barrier()

    def _wchunk(k, carry):
        rows = pl.ds(zbase + k * _ZCH, _ZCH)
        pltpu.sync_copy(partial.at[rows, :], out_hbm.at[c, rows, :])
        return carry
    lax.fori_loop(0, nz, _wchunk, 0)


_layer = pl.kernel(
    _layer_body,
    out_type=jax.ShapeDtypeStruct((_NC, _N, _D), jnp.float32),
    mesh=plsc.VectorSubcoreMesh(
        core_axis_name="c", subcore_axis_name="s",
        num_cores=_NC, num_subcores=_NS),
    scratch_types=[
        pltpu.VMEM_SHARED((_N, _D), jnp.float32),
        pltpu.VMEM((_C, _D), jnp.float32),
        pltpu.VMEM((_ZCH, _D), jnp.float32),
        pltpu.VMEM((_C,), jnp.int32),
        pltpu.VMEM((_C,), jnp.int32),
        pltpu.VMEM((_C,), jnp.float32),
    ],
)


@jax.jit
def _lgcn(x, src2d, dst2d, w2d):
    feats = [x]
    h = x
    for _ in range(3):
        p = _layer(h, src2d, dst2d, w2d)
        h = p[0] + p[1]
        feats.append(h)
    return jnp.concatenate(feats, axis=1)


def kernel(x, edge_index, edge_weight, edge_type):
    del edge_type
    src = edge_index[0].astype(jnp.int32).reshape(_ROWS, _C)
    dst = edge_index[1].astype(jnp.int32).reshape(_ROWS, _C)
    w = (0.5 * edge_weight.astype(jnp.float32)).reshape(_ROWS, _C)
    return _lgcn(x, src, dst, w)


# speedup vs baseline: 6.3424x; 6.3424x over previous
import functools

import jax
import jax.numpy as jnp
from jax import lax
from jax.experimental import pallas as pl
from jax.experimental.pallas import tpu as pltpu
from jax.experimental.pallas import tpu_sc as plsc

_N = 10000
_D = 128
_E = 320000
_C = 128
_ROWS = _E // _C
_NC = 2
_NS = 16
_NW = _NC * _NS
_RPW = _ROWS // _NW
_EXTRA = _ROWS - _RPW * _NW
_ZCH = 80
_LANES = 8


def _layer_body(h_hbm, src_hbm, dst_hbm, w_hbm, out_hbm,
                partial, gbuf, zbuf, src_v, dst_v, w_v):
    c = lax.axis_index("c")
    s = lax.axis_index("s")
    wid = s * _NC + c

    def _zrow(i, carry):
        for j in range(_LANES):
            zbuf[i, pl.ds(j * 16, 16)] = jnp.zeros((16,), jnp.float32)
        return carry
    lax.fori_loop(0, _ZCH, _zrow, 0)

    zbase = s * 640
    nz = jnp.where(s < _NS - 1, 8, 5)

    def _zchunk(k, carry):
        pltpu.sync_copy(zbuf, partial.at[pl.ds(zbase + k * _ZCH, _ZCH), :])
        return carry
    lax.fori_loop(0, nz, _zchunk, 0)

    plsc.subcore_barrier()

    nrows = _RPW + jnp.where(wid < _EXTRA, 1, 0)
    rbase = wid * _RPW + jnp.minimum(wid, _EXTRA)

    def _erow(r, carry):
        row = rbase + r
        pltpu.sync_copy(src_hbm.at[row], src_v)
        pltpu.sync_copy(dst_hbm.at[row], dst_v)
        pltpu.sync_copy(w_hbm.at[row], w_v.at[pl.ds(0, _C)])
        pltpu.sync_copy(h_hbm.at[src_v], gbuf)

        def _scale(e, carry2):
            w = w_v[pl.ds(e, 16)][0]
            for j in range(_LANES):
                gbuf[e, pl.ds(j * 16, 16)] = gbuf[e, pl.ds(j * 16, 16)] * w
            return carry2
        lax.fori_loop(0, _C, _scale, 0)

        pltpu.sync_copy(gbuf, partial.at[dst_v], add=True)
        return carry
    lax.fori_loop(0, nrows, _erow, 0)

    plsc.subcore_barrier()

    def _wchunk(k, carry):
        rows = pl.ds(zbase + k * _ZCH, _ZCH)
        pltpu.sync_copy(partial.at[rows, :], out_hbm.at[c, rows, :])
        return carry
    lax.fori_loop(0, nz, _wchunk, 0)


_layer = pl.kernel(
    _layer_body,
    out_type=jax.ShapeDtypeStruct((_NC, _N, _D), jnp.float32),
    mesh=plsc.VectorSubcoreMesh(
        core_axis_name="c", subcore_axis_name="s",
        num_cores=_NC, num_subcores=_NS),
    scratch_types=[
        pltpu.VMEM_SHARED((_N, _D), jnp.float32),
        pltpu.VMEM((_C, _D), jnp.float32),
        pltpu.VMEM((_ZCH, _D), jnp.float32),
        pltpu.VMEM((_C,), jnp.int32),
        pltpu.VMEM((_C,), jnp.int32),
        pltpu.VMEM((_C + 16,), jnp.float32),
    ],
)


@jax.jit
def _lgcn(x, src2d, dst2d, w2d):
    feats = [x]
    h = x
    for _ in range(3):
        p = _layer(h, src2d, dst2d, w2d)
        h = p[0] + p[1]
        feats.append(h)
    return jnp.concatenate(feats, axis=1)


def kernel(x, edge_index, edge_weight, edge_type):
    del edge_type
    src = edge_index[0].astype(jnp.int32).reshape(_ROWS, _C)
    dst = edge_index[1].astype(jnp.int32).reshape(_ROWS, _C)
    w = (0.5 * edge_weight.astype(jnp.float32)).reshape(_ROWS, _C)
    return _lgcn(x, src, dst, w)
